# baseline (speedup 1.0000x reference)
import jax
import jax.numpy as jnp
from jax import lax
from jax.experimental import pallas as pl
from jax.experimental.pallas import tpu as pltpu
from jax.experimental.pallas import tpu_sc as plsc

_DIM = 128
_SEQ = 1000

_ROWS_BIG = 64
_ROWS_LAST = _SEQ - 15 * _ROWS_BIG


def _do_chunk(x_hbm, wl_hbm, wp_hbm, out_hbm, idx_v, rows_v, pos_v, sem,
              base, nrows):
    pltpu.sync_copy(x_hbm.at[pl.ds(base, nrows)], idx_v.at[pl.ds(0, nrows)])
    gather = pltpu.async_copy(
        wl_hbm.at[idx_v.at[pl.ds(0, nrows)]], rows_v.at[pl.ds(0, nrows)], sem)
    pltpu.sync_copy(wp_hbm.at[pl.ds(base, nrows)], pos_v.at[pl.ds(0, nrows)])
    gather.wait()

    def add_row(r, carry):
        for c in range(_DIM // 16):
            sl = pl.ds(c * 16, 16)
            rows_v[r, sl] = rows_v[r, sl] + pos_v[r, sl]
        return carry

    lax.fori_loop(0, nrows, add_row, 0)
    pltpu.sync_copy(rows_v.at[pl.ds(0, nrows)], out_hbm.at[pl.ds(base, nrows)])


def _body(x_hbm, wl_hbm, wp_hbm, out_hbm, idx_v, rows_v, pos_v, sem):
    wid = lax.axis_index("s")

    @pl.when(wid < 15)
    def _():
        _do_chunk(x_hbm, wl_hbm, wp_hbm, out_hbm, idx_v, rows_v, pos_v, sem,
                  wid * _ROWS_BIG, _ROWS_BIG)

    @pl.when(wid == 15)
    def _():
        _do_chunk(x_hbm, wl_hbm, wp_hbm, out_hbm, idx_v, rows_v, pos_v, sem,
                  15 * _ROWS_BIG, _ROWS_LAST)


@jax.jit
def kernel(x, W_lettre, W_pos):
    mesh = plsc.VectorSubcoreMesh(core_axis_name="c", subcore_axis_name="s",
                                  num_cores=1)
    f = pl.kernel(
        _body,
        mesh=mesh,
        out_type=jax.ShapeDtypeStruct((_SEQ, _DIM), jnp.float32),
        scratch_types=[
            pltpu.VMEM((_ROWS_BIG,), jnp.int32),
            pltpu.VMEM((_ROWS_BIG, _DIM), jnp.float32),
            pltpu.VMEM((_ROWS_BIG, _DIM), jnp.float32),
            pltpu.SemaphoreType.DMA,
        ],
    )
    return f(x, W_lettre, W_pos)


# speedup vs baseline: 1.0095x; 1.0095x over previous
import jax
import jax.numpy as jnp
from jax import lax
from jax.experimental import pallas as pl
from jax.experimental.pallas import tpu as pltpu
from jax.experimental.pallas import tpu_sc as plsc

_DIM = 128
_SEQ = 1000

_ROWS_BIG = 64
_ROWS_LAST = _SEQ - 15 * _ROWS_BIG


def _do_chunk(x_hbm, wl_hbm, wp_hbm, out_hbm, idx_v, rows_v, pos_v, sem,
              sem_pos, base, nrows):
    pos_cp = pltpu.async_copy(
        wp_hbm.at[pl.ds(base, nrows)], pos_v.at[pl.ds(0, nrows)], sem_pos)
    pltpu.sync_copy(x_hbm.at[pl.ds(base, nrows)], idx_v.at[pl.ds(0, nrows)])
    gather = pltpu.async_copy(
        wl_hbm.at[idx_v.at[pl.ds(0, nrows)]], rows_v.at[pl.ds(0, nrows)], sem)
    pos_cp.wait()
    gather.wait()

    def add_row(r, carry):
        for c in range(_DIM // 16):
            sl = pl.ds(c * 16, 16)
            rows_v[r, sl] = rows_v[r, sl] + pos_v[r, sl]
        return carry

    lax.fori_loop(0, nrows, add_row, 0)
    pltpu.sync_copy(rows_v.at[pl.ds(0, nrows)], out_hbm.at[pl.ds(base, nrows)])


def _body(x_hbm, wl_hbm, wp_hbm, out_hbm, idx_v, rows_v, pos_v, sem, sem_pos):
    wid = lax.axis_index("s")

    @pl.when(wid < 15)
    def _():
        _do_chunk(x_hbm, wl_hbm, wp_hbm, out_hbm, idx_v, rows_v, pos_v, sem, sem_pos,
                  wid * _ROWS_BIG, _ROWS_BIG)

    @pl.when(wid == 15)
    def _():
        _do_chunk(x_hbm, wl_hbm, wp_hbm, out_hbm, idx_v, rows_v, pos_v, sem, sem_pos,
                  15 * _ROWS_BIG, _ROWS_LAST)


@jax.jit
def kernel(x, W_lettre, W_pos):
    mesh = plsc.VectorSubcoreMesh(core_axis_name="c", subcore_axis_name="s",
                                  num_cores=1)
    f = pl.kernel(
        _body,
        mesh=mesh,
        out_type=jax.ShapeDtypeStruct((_SEQ, _DIM), jnp.float32),
        scratch_types=[
            pltpu.VMEM((_ROWS_BIG,), jnp.int32),
            pltpu.VMEM((_ROWS_BIG, _DIM), jnp.float32),
            pltpu.VMEM((_ROWS_BIG, _DIM), jnp.float32),
            pltpu.SemaphoreType.DMA,
            pltpu.SemaphoreType.DMA,
        ],
    )
    return f(x, W_lettre, W_pos)
